# Initial kernel scaffold; baseline (speedup 1.0000x reference)
import jax
import jax.numpy as jnp
from jax.experimental import pallas as pl


def kernel(x, edge_index, batch, node_centrality, edge_centrality, W_l0, b_l0, W_r0, b_r0, W_l1, b_l1, W_r1, b_r1, Wc, bc):
    raise NotImplementedError("write your pallas kernel here")


# speedup vs baseline: 3.8563x; 3.8563x over previous
import functools

import jax
import jax.numpy as jnp
from jax import lax
from jax.experimental import pallas as pl
from jax.experimental.pallas import tpu as pltpu
from jax.experimental.pallas import tpu_sc as plsc

_N = 10000
_E = 320000
_D = 128
_G = 128
_OUT = 32
_NPAD = 10240

_NCORE = 2
_NSUB = 16
_NW = _NCORE * _NSUB
_EPW = _E // _NW
_K = 80
_NCHUNK = _EPW // _K
_EPT = _E // _NSUB
_KC = 128
_NCC = _EPT // _KC
_KCR = _EPT - _NCC * _KC
_RPT = _NPAD // _NSUB


@functools.cache
def _make_sc_agg(first_layer):
  mesh = plsc.VectorSubcoreMesh(
      core_axis_name="c", subcore_axis_name="s",
      num_cores=_NCORE, num_subcores=_NSUB)
  if first_layer:
    out_type = (jax.ShapeDtypeStruct((_NCORE, _NPAD, _D), jnp.float32),
                jax.ShapeDtypeStruct((_E,), jnp.float32))
  else:
    out_type = jax.ShapeDtypeStruct((_NCORE, _NPAD, _D), jnp.float32)

  scratch = [
      pltpu.VMEM_SHARED((_NPAD, _D), jnp.float32),
      pltpu.VMEM((_K,), jnp.int32),
      pltpu.VMEM((_K,), jnp.int32),
      pltpu.VMEM((_K,), jnp.float32),
      pltpu.VMEM((_K, _D), jnp.float32),
      pltpu.SemaphoreType.DMA,
  ]
  if first_layer:
    scratch += [
        pltpu.VMEM_SHARED((_NPAD,), jnp.float32),
        pltpu.VMEM((_NPAD,), jnp.float32),
        pltpu.VMEM((_KC,), jnp.int32),
        pltpu.VMEM((_KCR,), jnp.int32),
        pltpu.VMEM((_KC,), jnp.float32),
        pltpu.VMEM((_RPT,), jnp.float32),
    ]

  def body(y_hbm, src_hbm, dst_hbm, ec_hbm, *refs):
    if first_layer:
      (agg_out, fac_out, accum_sh, src_v, dst_v, fac_v, rows_v, sem,
       cnt_sh, cntf_v, dstc_v, dstr_v, ones_v, zflat_v) = refs
    else:
      (agg_out, accum_sh, src_v, dst_v, fac_v, rows_v, sem) = refs

    cid = lax.axis_index("c")
    sid = lax.axis_index("s")
    wid = sid * _NCORE + cid

    z16 = jnp.zeros((16,), jnp.float32)

    def _zr(i, c):
      for cc in range(_D // 16):
        rows_v[i, pl.ds(cc * 16, 16)] = z16
      return c
    lax.fori_loop(0, _K, _zr, 0)
    for b in range(_RPT // _K):
      pltpu.sync_copy(rows_v, accum_sh.at[pl.ds(sid * _RPT + b * _K, _K)])

    if first_layer:
      one16 = jnp.ones((16,), jnp.float32)
      for g in range(_KC // 16):
        ones_v[pl.ds(g * 16, 16)] = one16
      def _zf(i, c):
        zflat_v[pl.ds(i * 16, 16)] = z16
        return c
      lax.fori_loop(0, _RPT // 16, _zf, 0)
      pltpu.sync_copy(zflat_v, cnt_sh.at[pl.ds(sid * _RPT, _RPT)])

    plsc.subcore_barrier()

    if first_layer:
      cbase = sid * _EPT
      def _cchunk(k, c):
        pltpu.sync_copy(dst_hbm.at[pl.ds(cbase + k * _KC, _KC)], dstc_v)
        pltpu.sync_copy(ones_v, cnt_sh.at[dstc_v], add=True)
        return c
      lax.fori_loop(0, _NCC, _cchunk, 0)
      pltpu.sync_copy(dst_hbm.at[pl.ds(cbase + _NCC * _KC, _KCR)], dstr_v)
      pltpu.sync_copy(ones_v.at[pl.ds(0, _KCR)], cnt_sh.at[dstr_v], add=True)
      plsc.subcore_barrier()
      pltpu.sync_copy(cnt_sh, cntf_v)

    ebase = wid * _EPW
    def _chunk(ci, c):
      base = ebase + ci * _K
      pltpu.sync_copy(src_hbm.at[pl.ds(base, _K)], src_v)
      pltpu.sync_copy(dst_hbm.at[pl.ds(base, _K)], dst_v)
      pltpu.sync_copy(ec_hbm.at[pl.ds(base, _K)], fac_v)
      pltpu.async_copy(y_hbm.at[src_v], rows_v, sem).wait()
      if first_layer:
        for g in range(_K // 16):
          d16 = dst_v[pl.ds(g * 16, 16)]
          c16 = plsc.load_gather(cntf_v, [d16])
          e16 = fac_v[pl.ds(g * 16, 16)]
          fac_v[pl.ds(g * 16, 16)] = e16 / jnp.maximum(c16, 1.0)
        pltpu.sync_copy(fac_v, fac_out.at[pl.ds(base, _K)])
      for e in range(_K):
        bvec = plsc.load_gather(fac_v, [jnp.full((16,), e, jnp.int32)])
        for cc in range(_D // 16):
          rows_v[e, pl.ds(cc * 16, 16)] = rows_v[e, pl.ds(cc * 16, 16)] * bvec
      pltpu.sync_copy(rows_v, accum_sh.at[dst_v], add=True)
      return c
    lax.fori_loop(0, _NCHUNK, _chunk, 0)

    plsc.subcore_barrier()
    pltpu.sync_copy(accum_sh.at[pl.ds(sid * _RPT, _RPT)],
                    agg_out.at[cid, pl.ds(sid * _RPT, _RPT)])

  return pl.kernel(
      body, out_type=out_type, mesh=mesh, scratch_types=scratch,
      compiler_params=pltpu.CompilerParams(needs_layout_passes=False))


_BLK = 1024


def _layer_body(p_ref, x_ref, nc_ref, wl_ref, wr_ref, bl_ref, br_ref, o_ref):
  agg = p_ref[0] + p_ref[1]
  xr = x_ref[...] * nc_ref[...]
  acc = jnp.dot(agg, wl_ref[...], preferred_element_type=jnp.float32)
  acc = acc + jnp.dot(xr, wr_ref[...], preferred_element_type=jnp.float32)
  o_ref[...] = jnp.maximum(acc + bl_ref[...] + br_ref[...], 0.0)


def _layer_tc(p, xp, ncp, wlT, wrT, bl2, br2, interpret=False):
  return pl.pallas_call(
      _layer_body,
      grid=(_NPAD // _BLK,),
      in_specs=[
          pl.BlockSpec((_NCORE, _BLK, _D), lambda i: (0, i, 0)),
          pl.BlockSpec((_BLK, _D), lambda i: (i, 0)),
          pl.BlockSpec((_BLK, 1), lambda i: (i, 0)),
          pl.BlockSpec((_D, _D), lambda i: (0, 0)),
          pl.BlockSpec((_D, _D), lambda i: (0, 0)),
          pl.BlockSpec((1, _D), lambda i: (0, 0)),
          pl.BlockSpec((1, _D), lambda i: (0, 0)),
      ],
      out_specs=pl.BlockSpec((_BLK, _D), lambda i: (i, 0)),
      out_shape=jax.ShapeDtypeStruct((_NPAD, _D), jnp.float32),
      interpret=interpret,
  )(p, xp, ncp, wlT, wrT, bl2, br2)


def _pool_body(h_ref, b_ref, wc_ref, bc_ref, o_ref, ps_ref, pc_ref):
  i = pl.program_id(0)

  @pl.when(i == 0)
  def _():
    ps_ref[...] = jnp.zeros_like(ps_ref)
    pc_ref[...] = jnp.zeros_like(pc_ref)

  bvec = b_ref[...]
  iota = lax.broadcasted_iota(jnp.int32, (_G, _BLK), 0)
  oh = (iota == bvec).astype(jnp.float32)
  ps_ref[...] += jnp.dot(oh, h_ref[...], preferred_element_type=jnp.float32)
  pc_ref[...] += jnp.sum(oh, axis=1, keepdims=True)

  @pl.when(i == _NPAD // _BLK - 1)
  def _():
    pooled = ps_ref[...] / jnp.maximum(pc_ref[...], 1.0)
    o_ref[...] = (jnp.dot(pooled, wc_ref[...],
                          preferred_element_type=jnp.float32) + bc_ref[...])


def _pool_tc(h, batch2, wcT, bc2, interpret=False):
  return pl.pallas_call(
      _pool_body,
      grid=(_NPAD // _BLK,),
      in_specs=[
          pl.BlockSpec((_BLK, _D), lambda i: (i, 0)),
          pl.BlockSpec((1, _BLK), lambda i: (0, i)),
          pl.BlockSpec((_D, _OUT), lambda i: (0, 0)),
          pl.BlockSpec((1, _OUT), lambda i: (0, 0)),
      ],
      out_specs=pl.BlockSpec((_G, _OUT), lambda i: (0, 0)),
      out_shape=jax.ShapeDtypeStruct((_G, _OUT), jnp.float32),
      scratch_shapes=[
          pltpu.VMEM((_G, _D), jnp.float32),
          pltpu.VMEM((_G, 1), jnp.float32),
      ],
      interpret=interpret,
  )(h, batch2, wcT, bc2)


def kernel(x, edge_index, batch, node_centrality, edge_centrality,
           W_l0, b_l0, W_r0, b_r0, W_l1, b_l1, W_r1, b_r1, Wc, bc):
  f32 = jnp.float32
  xp = jnp.zeros((_NPAD, _D), f32).at[:_N].set(x)
  ncp = jnp.zeros((_NPAD, 1), f32).at[:_N, 0].set(node_centrality)
  batch2 = jnp.full((1, _NPAD), _G, jnp.int32).at[0, :_N].set(batch)

  src = edge_index[0]
  dst = edge_index[1]
  p0, fac = _make_sc_agg(True)(xp, src, dst, edge_centrality)
  h1 = _layer_tc(p0, xp, ncp, W_l0.T, W_r0.T,
                 b_l0.reshape(1, -1), b_r0.reshape(1, -1))
  p1 = _make_sc_agg(False)(h1, src, dst, fac)
  h2 = _layer_tc(p1, h1, ncp, W_l1.T, W_r1.T,
                 b_l1.reshape(1, -1), b_r1.reshape(1, -1))
  return _pool_tc(h2, batch2, Wc.T, bc.reshape(1, -1))


# speedup vs baseline: 5.4052x; 1.4016x over previous
import functools

import jax
import jax.numpy as jnp
from jax import lax
from jax.experimental import pallas as pl
from jax.experimental.pallas import tpu as pltpu
from jax.experimental.pallas import tpu_sc as plsc

_N = 10000
_E = 320000
_D = 128
_G = 128
_OUT = 32
_NPAD = 10240

_NCORE = 2
_NSUB = 16
_NW = _NCORE * _NSUB
_EPW = _E // _NW
_K = 80
_NCHUNK = _EPW // _K
_RING = 4
_NSUP = (_NCHUNK - 1) // _RING
_EPAD = 8 * _K
_EPT = _E // _NSUB
_KC = 128
_NCC = _EPT // _KC
_KCR = _EPT - _NCC * _KC
_RPT = _NPAD // _NSUB


def _sc_mesh():
  return plsc.VectorSubcoreMesh(
      core_axis_name="c", subcore_axis_name="s",
      num_cores=_NCORE, num_subcores=_NSUB)


@functools.cache
def _make_sc_first():
  out_type = (jax.ShapeDtypeStruct((_NCORE, _NPAD, _D), jnp.float32),
              jax.ShapeDtypeStruct((_E,), jnp.float32))

  scratch = [
      pltpu.VMEM_SHARED((_NPAD, _D), jnp.float32),
      pltpu.VMEM((_K,), jnp.int32),
      pltpu.VMEM((_K,), jnp.int32),
      pltpu.VMEM((_K,), jnp.float32),
      pltpu.VMEM((_K, _D), jnp.float32),
      pltpu.SemaphoreType.DMA,
      pltpu.VMEM_SHARED((_NPAD,), jnp.float32),
      pltpu.VMEM((_NPAD,), jnp.float32),
      pltpu.VMEM((_KC,), jnp.int32),
      pltpu.VMEM((_KCR,), jnp.int32),
      pltpu.VMEM((_KC,), jnp.float32),
      pltpu.VMEM((_RPT,), jnp.float32),
  ]

  def body(y_hbm, src_hbm, dst_hbm, ec_hbm, agg_out, fac_out,
           accum_sh, src_v, dst_v, fac_v, rows_v, sem,
           cnt_sh, cntf_v, dstc_v, dstr_v, ones_v, zflat_v):
    cid = lax.axis_index("c")
    sid = lax.axis_index("s")
    wid = sid * _NCORE + cid

    z16 = jnp.zeros((16,), jnp.float32)

    def _zr(i, c):
      for cc in range(_D // 16):
        rows_v[i, pl.ds(cc * 16, 16)] = z16
      return c
    lax.fori_loop(0, _K, _zr, 0)
    for b in range(_RPT // _K):
      pltpu.sync_copy(rows_v, accum_sh.at[pl.ds(sid * _RPT + b * _K, _K)])

    one16 = jnp.ones((16,), jnp.float32)
    for g in range(_KC // 16):
      ones_v[pl.ds(g * 16, 16)] = one16
    def _zf(i, c):
      zflat_v[pl.ds(i * 16, 16)] = z16
      return c
    lax.fori_loop(0, _RPT // 16, _zf, 0)
    pltpu.sync_copy(zflat_v, cnt_sh.at[pl.ds(sid * _RPT, _RPT)])

    plsc.subcore_barrier()

    cbase = sid * _EPT
    def _cchunk(k, c):
      pltpu.sync_copy(dst_hbm.at[pl.ds(cbase + k * _KC, _KC)], dstc_v)
      pltpu.sync_copy(ones_v, cnt_sh.at[dstc_v], add=True)
      return c
    lax.fori_loop(0, _NCC, _cchunk, 0)
    pltpu.sync_copy(dst_hbm.at[pl.ds(cbase + _NCC * _KC, _KCR)], dstr_v)
    pltpu.sync_copy(ones_v.at[pl.ds(0, _KCR)], cnt_sh.at[dstr_v], add=True)
    plsc.subcore_barrier()
    pltpu.sync_copy(cnt_sh, cntf_v)

    ebase = wid * _EPW
    def _chunk(ci, c):
      base = ebase + ci * _K
      pltpu.sync_copy(src_hbm.at[pl.ds(base, _K)], src_v)
      pltpu.sync_copy(dst_hbm.at[pl.ds(base, _K)], dst_v)
      pltpu.sync_copy(ec_hbm.at[pl.ds(base, _K)], fac_v)
      pltpu.async_copy(y_hbm.at[src_v], rows_v, sem).wait()
      for g in range(_K // 16):
        d16 = dst_v[pl.ds(g * 16, 16)]
        c16 = plsc.load_gather(cntf_v, [d16])
        e16 = fac_v[pl.ds(g * 16, 16)]
        fac_v[pl.ds(g * 16, 16)] = e16 / jnp.maximum(c16, 1.0)
      pltpu.sync_copy(fac_v, fac_out.at[pl.ds(base, _K)])
      for e in range(_K):
        bvec = plsc.load_gather(fac_v, [jnp.full((16,), e, jnp.int32)])
        for cc in range(_D // 16):
          rows_v[e, pl.ds(cc * 16, 16)] = rows_v[e, pl.ds(cc * 16, 16)] * bvec
      pltpu.sync_copy(rows_v, accum_sh.at[dst_v], add=True)
      return c
    lax.fori_loop(0, _NCHUNK, _chunk, 0)

    plsc.subcore_barrier()
    pltpu.sync_copy(accum_sh.at[pl.ds(sid * _RPT, _RPT)],
                    agg_out.at[cid, pl.ds(sid * _RPT, _RPT)])

  return pl.kernel(
      body, out_type=out_type, mesh=_sc_mesh(), scratch_types=scratch,
      compiler_params=pltpu.CompilerParams(needs_layout_passes=False))


@functools.cache
def _make_sc_next():
  out_type = jax.ShapeDtypeStruct((_NCORE, _NPAD, _D), jnp.float32)

  scratch = [pltpu.VMEM_SHARED((_NPAD, _D), jnp.float32)]
  scratch += [pltpu.VMEM((_K, _D), jnp.float32) for _ in range(_RING)]
  scratch += [pltpu.VMEM((_K,), jnp.int32) for _ in range(_RING)]
  scratch += [pltpu.VMEM((_K,), jnp.int32) for _ in range(_RING)]
  scratch += [pltpu.VMEM((_K,), jnp.float32) for _ in range(_RING)]
  scratch += [pltpu.SemaphoreType.DMA for _ in range(3 * _RING)]

  def body(y_hbm, src_hbm, dst_hbm, fac_hbm, agg_out, *refs):
    rest = list(refs)
    accum_sh = rest.pop(0)
    rows = [rest.pop(0) for _ in range(_RING)]
    srcs = [rest.pop(0) for _ in range(_RING)]
    dsts = [rest.pop(0) for _ in range(_RING)]
    facs = [rest.pop(0) for _ in range(_RING)]
    isem = [rest.pop(0) for _ in range(_RING)]
    gsem = [rest.pop(0) for _ in range(_RING)]
    ssem = [rest.pop(0) for _ in range(_RING)]

    cid = lax.axis_index("c")
    sid = lax.axis_index("s")
    wid = sid * _NCORE + cid
    ebase = wid * _EPW

    z16 = jnp.zeros((16,), jnp.float32)
    zi16 = jnp.zeros((16,), jnp.int32)

    def _zr(i, c):
      for cc in range(_D // 16):
        rows[0][i, pl.ds(cc * 16, 16)] = z16
        rows[2][i, pl.ds(cc * 16, 16)] = z16
        rows[3][i, pl.ds(cc * 16, 16)] = z16
      return c
    lax.fori_loop(0, _K, _zr, 0)
    for g in range(_K // 16):
      dsts[2][pl.ds(g * 16, 16)] = zi16
      dsts[3][pl.ds(g * 16, 16)] = zi16
    for b in range(_RPT // _K):
      pltpu.sync_copy(rows[0], accum_sh.at[pl.ds(sid * _RPT + b * _K, _K)])

    plsc.subcore_barrier()

    def _start_idx(ch, p):
      base = ebase + ch * _K
      pltpu.async_copy(src_hbm.at[pl.ds(base, _K)], srcs[p], isem[p])
      pltpu.async_copy(dst_hbm.at[pl.ds(base, _K)], dsts[p], isem[p])
      pltpu.async_copy(fac_hbm.at[pl.ds(base, _K)], facs[p], isem[p])

    def _wait_idx(ch, p):
      base = ebase + ch * _K
      pltpu.make_async_copy(
          src_hbm.at[pl.ds(base, _K)], srcs[p], isem[p]).wait()
      pltpu.make_async_copy(
          dst_hbm.at[pl.ds(base, _K)], dsts[p], isem[p]).wait()
      pltpu.make_async_copy(
          fac_hbm.at[pl.ds(base, _K)], facs[p], isem[p]).wait()

    def _start_gather(p):
      pltpu.async_copy(y_hbm.at[srcs[p]], rows[p], gsem[p])

    def _start_scatter(p):
      pltpu.async_copy(rows[p], accum_sh.at[dsts[p]], ssem[p], add=True)

    def _wait_scatter(p):
      pltpu.make_async_copy(rows[p], accum_sh.at[dsts[p]], ssem[p]).wait()

    def _process(ch, p):
      pltpu.make_async_copy(y_hbm.at[srcs[p]], rows[p], gsem[p]).wait()
      rows_p, fac_p = rows[p], facs[p]
      def _scg(g, c2):
        for j in range(16):
          e = g * 16 + j
          bvec = plsc.load_gather(fac_p, [jnp.full((16,), e, jnp.int32)])
          for cc in range(_D // 16):
            rows_p[e, pl.ds(cc * 16, 16)] = (
                rows_p[e, pl.ds(cc * 16, 16)] * bvec)
        return c2
      lax.fori_loop(0, _K // 16, _scg, 0)
      _start_scatter(p)

    _start_idx(0, 0)
    _wait_idx(0, 0)
    _start_gather(0)
    _start_idx(1, 1)
    _start_scatter(2)
    _start_scatter(3)

    def _miter(i, c):
      for b in range(_RING):
        ch = _RING * i + b
        p1 = (b + 1) % _RING
        p2 = (b + 2) % _RING
        _wait_idx(ch + 1, p1)
        _start_gather(p1)
        _wait_scatter(p2)
        _start_idx(ch + 2, p2)
        _process(ch, b)
      return c
    lax.fori_loop(0, _NSUP, _miter, 0)

    _wait_scatter(2)
    _process(_NCHUNK - 1, 0)
    _wait_idx(_NCHUNK + 1, 1)
    _wait_scatter(3)
    _wait_scatter(0)

    plsc.subcore_barrier()
    pltpu.sync_copy(accum_sh.at[pl.ds(sid * _RPT, _RPT)],
                    agg_out.at[cid, pl.ds(sid * _RPT, _RPT)])

  return pl.kernel(
      body, out_type=out_type, mesh=_sc_mesh(), scratch_types=scratch,
      compiler_params=pltpu.CompilerParams(needs_layout_passes=False))


_BLK = 1024


def _layer_body(p_ref, x_ref, nc_ref, wl_ref, wr_ref, bl_ref, br_ref, o_ref):
  agg = p_ref[0] + p_ref[1]
  xr = x_ref[...] * nc_ref[...]
  acc = jnp.dot(agg, wl_ref[...], preferred_element_type=jnp.float32)
  acc = acc + jnp.dot(xr, wr_ref[...], preferred_element_type=jnp.float32)
  o_ref[...] = jnp.maximum(acc + bl_ref[...] + br_ref[...], 0.0)


def _layer_tc(p, xp, ncp, wlT, wrT, bl2, br2, interpret=False):
  return pl.pallas_call(
      _layer_body,
      grid=(_NPAD // _BLK,),
      in_specs=[
          pl.BlockSpec((_NCORE, _BLK, _D), lambda i: (0, i, 0)),
          pl.BlockSpec((_BLK, _D), lambda i: (i, 0)),
          pl.BlockSpec((_BLK, 1), lambda i: (i, 0)),
          pl.BlockSpec((_D, _D), lambda i: (0, 0)),
          pl.BlockSpec((_D, _D), lambda i: (0, 0)),
          pl.BlockSpec((1, _D), lambda i: (0, 0)),
          pl.BlockSpec((1, _D), lambda i: (0, 0)),
      ],
      out_specs=pl.BlockSpec((_BLK, _D), lambda i: (i, 0)),
      out_shape=jax.ShapeDtypeStruct((_NPAD, _D), jnp.float32),
      interpret=interpret,
  )(p, xp, ncp, wlT, wrT, bl2, br2)


def _pool_body(h_ref, b_ref, wc_ref, bc_ref, o_ref, ps_ref, pc_ref):
  i = pl.program_id(0)

  @pl.when(i == 0)
  def _():
    ps_ref[...] = jnp.zeros_like(ps_ref)
    pc_ref[...] = jnp.zeros_like(pc_ref)

  bvec = b_ref[...]
  iota = lax.broadcasted_iota(jnp.int32, (_G, _BLK), 0)
  oh = (iota == bvec).astype(jnp.float32)
  ps_ref[...] += jnp.dot(oh, h_ref[...], preferred_element_type=jnp.float32)
  pc_ref[...] += jnp.sum(oh, axis=1, keepdims=True)

  @pl.when(i == _NPAD // _BLK - 1)
  def _():
    pooled = ps_ref[...] / jnp.maximum(pc_ref[...], 1.0)
    o_ref[...] = (jnp.dot(pooled, wc_ref[...],
                          preferred_element_type=jnp.float32) + bc_ref[...])


def _pool_tc(h, batch2, wcT, bc2, interpret=False):
  return pl.pallas_call(
      _pool_body,
      grid=(_NPAD // _BLK,),
      in_specs=[
          pl.BlockSpec((_BLK, _D), lambda i: (i, 0)),
          pl.BlockSpec((1, _BLK), lambda i: (0, i)),
          pl.BlockSpec((_D, _OUT), lambda i: (0, 0)),
          pl.BlockSpec((1, _OUT), lambda i: (0, 0)),
      ],
      out_specs=pl.BlockSpec((_G, _OUT), lambda i: (0, 0)),
      out_shape=jax.ShapeDtypeStruct((_G, _OUT), jnp.float32),
      scratch_shapes=[
          pltpu.VMEM((_G, _D), jnp.float32),
          pltpu.VMEM((_G, 1), jnp.float32),
      ],
      interpret=interpret,
  )(h, batch2, wcT, bc2)


def kernel(x, edge_index, batch, node_centrality, edge_centrality,
           W_l0, b_l0, W_r0, b_r0, W_l1, b_l1, W_r1, b_r1, Wc, bc):
  f32 = jnp.float32
  xp = jnp.zeros((_NPAD, _D), f32).at[:_N].set(x)
  ncp = jnp.zeros((_NPAD, 1), f32).at[:_N, 0].set(node_centrality)
  batch2 = jnp.full((1, _NPAD), _G, jnp.int32).at[0, :_N].set(batch)

  src = edge_index[0]
  dst = edge_index[1]
  zpad_i = jnp.zeros((_EPAD,), jnp.int32)
  srcp = jnp.concatenate([src, zpad_i])
  dstp = jnp.concatenate([dst, zpad_i])

  p0, fac = _make_sc_first()(xp, src, dst, edge_centrality)
  h1 = _layer_tc(p0, xp, ncp, W_l0.T, W_r0.T,
                 b_l0.reshape(1, -1), b_r0.reshape(1, -1))
  facp = jnp.concatenate([fac, jnp.zeros((_EPAD,), f32)])
  p1 = _make_sc_next()(h1, srcp, dstp, facp)
  h2 = _layer_tc(p1, h1, ncp, W_l1.T, W_r1.T,
                 b_l1.reshape(1, -1), b_r1.reshape(1, -1))
  return _pool_tc(h2, batch2, Wc.T, bc.reshape(1, -1))


# speedup vs baseline: 7.3815x; 1.3656x over previous
import functools

import jax
import jax.numpy as jnp
from jax import lax
from jax.experimental import pallas as pl
from jax.experimental.pallas import tpu as pltpu
from jax.experimental.pallas import tpu_sc as plsc

_N = 10000
_E = 320000
_D = 128
_G = 128
_OUT = 32
_NPAD = 10240

_NCORE = 2
_NSUB = 16
_NW = _NCORE * _NSUB
_EPW = _E // _NW
_K = 80
_NCHUNK = _EPW // _K
_RING = 4
_NSUP = (_NCHUNK - 1) // _RING
_EPAD = 8 * _K
_EPT = _E // _NSUB
_KC = 128
_NCC = _EPT // _KC
_KCR = _EPT - _NCC * _KC
_RPT = _NPAD // _NSUB


def _sc_mesh():
  return plsc.VectorSubcoreMesh(
      core_axis_name="c", subcore_axis_name="s",
      num_cores=_NCORE, num_subcores=_NSUB)


@functools.cache
def _make_sc_fac():
  out_type = jax.ShapeDtypeStruct((_E + _EPAD,), jnp.float32)

  scratch = [
      pltpu.VMEM_SHARED((_NPAD,), jnp.float32),
      pltpu.VMEM((_NPAD,), jnp.float32),
      pltpu.VMEM((_K,), jnp.int32),
      pltpu.VMEM((_K,), jnp.float32),
      pltpu.VMEM((_KC,), jnp.int32),
      pltpu.VMEM((_KCR,), jnp.int32),
      pltpu.VMEM((_KC,), jnp.float32),
      pltpu.VMEM((_RPT,), jnp.float32),
  ]

  def body(dst_hbm, ec_hbm, fac_out,
           cnt_sh, cntf_v, dst_v, fac_v, dstc_v, dstr_v, ones_v, zflat_v):
    cid = lax.axis_index("c")
    sid = lax.axis_index("s")
    wid = sid * _NCORE + cid

    z16 = jnp.zeros((16,), jnp.float32)
    one16 = jnp.ones((16,), jnp.float32)
    for g in range(_KC // 16):
      ones_v[pl.ds(g * 16, 16)] = one16
    def _zf(i, c):
      zflat_v[pl.ds(i * 16, 16)] = z16
      return c
    lax.fori_loop(0, _RPT // 16, _zf, 0)
    pltpu.sync_copy(zflat_v, cnt_sh.at[pl.ds(sid * _RPT, _RPT)])
    plsc.subcore_barrier()

    cbase = sid * _EPT
    def _cchunk(k, c):
      pltpu.sync_copy(dst_hbm.at[pl.ds(cbase + k * _KC, _KC)], dstc_v)
      pltpu.sync_copy(ones_v, cnt_sh.at[dstc_v], add=True)
      return c
    lax.fori_loop(0, _NCC, _cchunk, 0)
    pltpu.sync_copy(dst_hbm.at[pl.ds(cbase + _NCC * _KC, _KCR)], dstr_v)
    pltpu.sync_copy(ones_v.at[pl.ds(0, _KCR)], cnt_sh.at[dstr_v], add=True)
    plsc.subcore_barrier()
    pltpu.sync_copy(cnt_sh, cntf_v)

    ebase = wid * _EPW
    def _chunk(ci, c):
      base = ebase + ci * _K
      pltpu.sync_copy(dst_hbm.at[pl.ds(base, _K)], dst_v)
      pltpu.sync_copy(ec_hbm.at[pl.ds(base, _K)], fac_v)
      for g in range(_K // 16):
        d16 = dst_v[pl.ds(g * 16, 16)]
        c16 = plsc.load_gather(cntf_v, [d16])
        e16 = fac_v[pl.ds(g * 16, 16)]
        fac_v[pl.ds(g * 16, 16)] = e16 / jnp.maximum(c16, 1.0)
      pltpu.sync_copy(fac_v, fac_out.at[pl.ds(base, _K)])
      return c
    lax.fori_loop(0, _NCHUNK, _chunk, 0)

  return pl.kernel(
      body, out_type=out_type, mesh=_sc_mesh(), scratch_types=scratch,
      compiler_params=pltpu.CompilerParams(needs_layout_passes=False))


@functools.cache
def _make_sc_next():
  out_type = jax.ShapeDtypeStruct((_NCORE, _NPAD, _D), jnp.float32)

  scratch = [pltpu.VMEM_SHARED((_NPAD, _D), jnp.float32)]
  scratch += [pltpu.VMEM((_K, _D), jnp.float32) for _ in range(_RING)]
  scratch += [pltpu.VMEM((_K,), jnp.int32) for _ in range(_RING)]
  scratch += [pltpu.VMEM((_K,), jnp.int32) for _ in range(_RING)]
  scratch += [pltpu.VMEM((_K,), jnp.float32) for _ in range(_RING)]
  scratch += [pltpu.SemaphoreType.DMA for _ in range(3 * _RING)]

  def body(y_hbm, src_hbm, dst_hbm, fac_hbm, agg_out, *refs):
    rest = list(refs)
    accum_sh = rest.pop(0)
    rows = [rest.pop(0) for _ in range(_RING)]
    srcs = [rest.pop(0) for _ in range(_RING)]
    dsts = [rest.pop(0) for _ in range(_RING)]
    facs = [rest.pop(0) for _ in range(_RING)]
    isem = [rest.pop(0) for _ in range(_RING)]
    gsem = [rest.pop(0) for _ in range(_RING)]
    ssem = [rest.pop(0) for _ in range(_RING)]

    cid = lax.axis_index("c")
    sid = lax.axis_index("s")
    wid = sid * _NCORE + cid
    ebase = wid * _EPW

    z16 = jnp.zeros((16,), jnp.float32)
    zi16 = jnp.zeros((16,), jnp.int32)

    def _zr(i, c):
      for cc in range(_D // 16):
        rows[0][i, pl.ds(cc * 16, 16)] = z16
        rows[2][i, pl.ds(cc * 16, 16)] = z16
        rows[3][i, pl.ds(cc * 16, 16)] = z16
      return c
    lax.fori_loop(0, _K, _zr, 0)
    for g in range(_K // 16):
      dsts[2][pl.ds(g * 16, 16)] = zi16
      dsts[3][pl.ds(g * 16, 16)] = zi16
    for b in range(_RPT // _K):
      pltpu.sync_copy(rows[0], accum_sh.at[pl.ds(sid * _RPT + b * _K, _K)])

    plsc.subcore_barrier()

    def _start_idx(ch, p):
      base = ebase + ch * _K
      pltpu.async_copy(src_hbm.at[pl.ds(base, _K)], srcs[p], isem[p])
      pltpu.async_copy(dst_hbm.at[pl.ds(base, _K)], dsts[p], isem[p])
      pltpu.async_copy(fac_hbm.at[pl.ds(base, _K)], facs[p], isem[p])

    def _wait_idx(ch, p):
      base = ebase + ch * _K
      pltpu.make_async_copy(
          src_hbm.at[pl.ds(base, _K)], srcs[p], isem[p]).wait()
      pltpu.make_async_copy(
          dst_hbm.at[pl.ds(base, _K)], dsts[p], isem[p]).wait()
      pltpu.make_async_copy(
          fac_hbm.at[pl.ds(base, _K)], facs[p], isem[p]).wait()

    def _start_gather(p):
      pltpu.async_copy(y_hbm.at[srcs[p]], rows[p], gsem[p])

    def _start_scatter(p):
      pltpu.async_copy(rows[p], accum_sh.at[dsts[p]], ssem[p], add=True)

    def _wait_scatter(p):
      pltpu.make_async_copy(rows[p], accum_sh.at[dsts[p]], ssem[p]).wait()

    def _process(ch, p):
      pltpu.make_async_copy(y_hbm.at[srcs[p]], rows[p], gsem[p]).wait()
      rows_p, fac_p = rows[p], facs[p]
      def _scg(g, c2):
        for j in range(16):
          e = g * 16 + j
          bvec = plsc.load_gather(fac_p, [jnp.full((16,), e, jnp.int32)])
          for cc in range(_D // 16):
            rows_p[e, pl.ds(cc * 16, 16)] = (
                rows_p[e, pl.ds(cc * 16, 16)] * bvec)
        return c2
      lax.fori_loop(0, _K // 16, _scg, 0)
      _start_scatter(p)

    _start_idx(0, 0)
    _wait_idx(0, 0)
    _start_gather(0)
    _start_idx(1, 1)
    _start_scatter(2)
    _start_scatter(3)

    def _miter(i, c):
      for b in range(_RING):
        ch = _RING * i + b
        p1 = (b + 1) % _RING
        p2 = (b + 2) % _RING
        _wait_idx(ch + 1, p1)
        _start_gather(p1)
        _wait_scatter(p2)
        _start_idx(ch + 2, p2)
        _process(ch, b)
      return c
    lax.fori_loop(0, _NSUP, _miter, 0)

    _wait_scatter(2)
    _process(_NCHUNK - 1, 0)
    _wait_idx(_NCHUNK + 1, 1)
    _wait_scatter(3)
    _wait_scatter(0)

    plsc.subcore_barrier()
    pltpu.sync_copy(accum_sh.at[pl.ds(sid * _RPT, _RPT)],
                    agg_out.at[cid, pl.ds(sid * _RPT, _RPT)])

  return pl.kernel(
      body, out_type=out_type, mesh=_sc_mesh(), scratch_types=scratch,
      compiler_params=pltpu.CompilerParams(needs_layout_passes=False))


_BLK = 1024


def _layer_body(p_ref, x_ref, nc_ref, wl_ref, wr_ref, bl_ref, br_ref, o_ref):
  agg = p_ref[0] + p_ref[1]
  xr = x_ref[...] * nc_ref[...]
  acc = jnp.dot(agg, wl_ref[...], preferred_element_type=jnp.float32)
  acc = acc + jnp.dot(xr, wr_ref[...], preferred_element_type=jnp.float32)
  o_ref[...] = jnp.maximum(acc + bl_ref[...] + br_ref[...], 0.0)


def _layer_tc(p, xp, ncp, wlT, wrT, bl2, br2, interpret=False):
  return pl.pallas_call(
      _layer_body,
      grid=(_NPAD // _BLK,),
      in_specs=[
          pl.BlockSpec((_NCORE, _BLK, _D), lambda i: (0, i, 0)),
          pl.BlockSpec((_BLK, _D), lambda i: (i, 0)),
          pl.BlockSpec((_BLK, 1), lambda i: (i, 0)),
          pl.BlockSpec((_D, _D), lambda i: (0, 0)),
          pl.BlockSpec((_D, _D), lambda i: (0, 0)),
          pl.BlockSpec((1, _D), lambda i: (0, 0)),
          pl.BlockSpec((1, _D), lambda i: (0, 0)),
      ],
      out_specs=pl.BlockSpec((_BLK, _D), lambda i: (i, 0)),
      out_shape=jax.ShapeDtypeStruct((_NPAD, _D), jnp.float32),
      interpret=interpret,
  )(p, xp, ncp, wlT, wrT, bl2, br2)


def _pool_body(h_ref, b_ref, wc_ref, bc_ref, o_ref, ps_ref, pc_ref):
  i = pl.program_id(0)

  @pl.when(i == 0)
  def _():
    ps_ref[...] = jnp.zeros_like(ps_ref)
    pc_ref[...] = jnp.zeros_like(pc_ref)

  bvec = b_ref[...]
  iota = lax.broadcasted_iota(jnp.int32, (_G, _BLK), 0)
  oh = (iota == bvec).astype(jnp.float32)
  ps_ref[...] += jnp.dot(oh, h_ref[...], preferred_element_type=jnp.float32)
  pc_ref[...] += jnp.sum(oh, axis=1, keepdims=True)

  @pl.when(i == _NPAD // _BLK - 1)
  def _():
    pooled = ps_ref[...] / jnp.maximum(pc_ref[...], 1.0)
    o_ref[...] = (jnp.dot(pooled, wc_ref[...],
                          preferred_element_type=jnp.float32) + bc_ref[...])


def _pool_tc(h, batch2, wcT, bc2, interpret=False):
  return pl.pallas_call(
      _pool_body,
      grid=(_NPAD // _BLK,),
      in_specs=[
          pl.BlockSpec((_BLK, _D), lambda i: (i, 0)),
          pl.BlockSpec((1, _BLK), lambda i: (0, i)),
          pl.BlockSpec((_D, _OUT), lambda i: (0, 0)),
          pl.BlockSpec((1, _OUT), lambda i: (0, 0)),
      ],
      out_specs=pl.BlockSpec((_G, _OUT), lambda i: (0, 0)),
      out_shape=jax.ShapeDtypeStruct((_G, _OUT), jnp.float32),
      scratch_shapes=[
          pltpu.VMEM((_G, _D), jnp.float32),
          pltpu.VMEM((_G, 1), jnp.float32),
      ],
      interpret=interpret,
  )(h, batch2, wcT, bc2)


def kernel(x, edge_index, batch, node_centrality, edge_centrality,
           W_l0, b_l0, W_r0, b_r0, W_l1, b_l1, W_r1, b_r1, Wc, bc):
  f32 = jnp.float32
  xp = jnp.zeros((_NPAD, _D), f32).at[:_N].set(x)
  ncp = jnp.zeros((_NPAD, 1), f32).at[:_N, 0].set(node_centrality)
  batch2 = jnp.full((1, _NPAD), _G, jnp.int32).at[0, :_N].set(batch)

  src = edge_index[0]
  dst = edge_index[1]
  zpad_i = jnp.zeros((_EPAD,), jnp.int32)
  srcp = jnp.concatenate([src, zpad_i])
  dstp = jnp.concatenate([dst, zpad_i])

  facp = _make_sc_fac()(dst, edge_centrality)
  p0 = _make_sc_next()(xp, srcp, dstp, facp)
  h1 = _layer_tc(p0, xp, ncp, W_l0.T, W_r0.T,
                 b_l0.reshape(1, -1), b_r0.reshape(1, -1))
  p1 = _make_sc_next()(h1, srcp, dstp, facp)
  h2 = _layer_tc(p1, h1, ncp, W_l1.T, W_r1.T,
                 b_l1.reshape(1, -1), b_r1.reshape(1, -1))
  return _pool_tc(h2, batch2, Wc.T, bc.reshape(1, -1))


# speedup vs baseline: 9.6562x; 1.3082x over previous
import functools

import jax
import jax.numpy as jnp
from jax import lax
from jax.experimental import pallas as pl
from jax.experimental.pallas import tpu as pltpu
from jax.experimental.pallas import tpu_sc as plsc

_N = 10000
_E = 320000
_D = 128
_G = 128
_OUT = 32
_NPAD = 10240

_NCORE = 2
_NSUB = 16
_NW = _NCORE * _NSUB
_EPW = _E // _NW
_K = 80
_NCHUNK = _EPW // _K
_RING = 4
_NSUP = (_NCHUNK - 1) // _RING
_EPAD = 8 * _K
_EPT = _E // _NSUB
_KC = 128
_NCC = _EPT // _KC
_KCR = _EPT - _NCC * _KC
_RPT = _NPAD // _NSUB


def _sc_mesh():
  return plsc.VectorSubcoreMesh(
      core_axis_name="c", subcore_axis_name="s",
      num_cores=_NCORE, num_subcores=_NSUB)


@functools.cache
def _make_sc_fac():
  out_type = jax.ShapeDtypeStruct((_E + _EPAD,), jnp.float32)

  scratch = [pltpu.VMEM_SHARED((_NPAD,), jnp.float32)]
  scratch += [pltpu.VMEM((_NPAD,), jnp.float32)]
  scratch += [pltpu.VMEM((_K,), jnp.int32) for _ in range(4)]
  scratch += [pltpu.VMEM((_K,), jnp.float32) for _ in range(4)]
  scratch += [pltpu.SemaphoreType.DMA for _ in range(4)]
  scratch += [
      pltpu.VMEM((_KC,), jnp.int32),
      pltpu.VMEM((_KC,), jnp.int32),
      pltpu.VMEM((_KCR,), jnp.int32),
      pltpu.VMEM((_KC,), jnp.float32),
      pltpu.VMEM((_RPT,), jnp.float32),
      pltpu.SemaphoreType.DMA,
      pltpu.SemaphoreType.DMA,
  ]

  def body(dst_hbm, ec_hbm, fac_out, *refs):
    rest = list(refs)
    cnt_sh = rest.pop(0)
    cntf_v = rest.pop(0)
    dsts = [rest.pop(0) for _ in range(4)]
    facs = [rest.pop(0) for _ in range(4)]
    isem = [rest.pop(0) for _ in range(4)]
    (dstc0, dstc1, dstr_v, ones_v, zflat_v, cs0, cs1) = rest

    cid = lax.axis_index("c")
    sid = lax.axis_index("s")
    wid = sid * _NCORE + cid

    z16 = jnp.zeros((16,), jnp.float32)
    one16 = jnp.ones((16,), jnp.float32)
    for g in range(_KC // 16):
      ones_v[pl.ds(g * 16, 16)] = one16
    def _zf(i, c):
      zflat_v[pl.ds(i * 16, 16)] = z16
      return c
    lax.fori_loop(0, _RPT // 16, _zf, 0)
    pltpu.sync_copy(zflat_v, cnt_sh.at[pl.ds(sid * _RPT, _RPT)])
    plsc.subcore_barrier()

    cbase = sid * _EPT
    def _cload(k, buf, sem):
      pltpu.async_copy(dst_hbm.at[pl.ds(cbase + k * _KC, _KC)], buf, sem)
    def _cwait(k, buf, sem):
      pltpu.make_async_copy(
          dst_hbm.at[pl.ds(cbase + k * _KC, _KC)], buf, sem).wait()
    _cload(0, dstc0, cs0)
    _cload(1, dstc1, cs1)
    def _citer(i, c):
      k0 = 2 * i
      _cwait(k0, dstc0, cs0)
      pltpu.sync_copy(ones_v, cnt_sh.at[dstc0], add=True)
      _cload(k0 + 2, dstc0, cs0)
      _cwait(k0 + 1, dstc1, cs1)
      pltpu.sync_copy(ones_v, cnt_sh.at[dstc1], add=True)
      _cload(k0 + 3, dstc1, cs1)
      return c
    lax.fori_loop(0, _NCC // 2, _citer, 0)
    _cwait(_NCC, dstc0, cs0)
    _cwait(_NCC + 1, dstc1, cs1)
    pltpu.sync_copy(dst_hbm.at[pl.ds(cbase + _NCC * _KC, _KCR)], dstr_v)
    pltpu.sync_copy(ones_v.at[pl.ds(0, _KCR)], cnt_sh.at[dstr_v], add=True)
    plsc.subcore_barrier()
    pltpu.sync_copy(cnt_sh, cntf_v)

    ebase = wid * _EPW

    def _start_idx(ch, p):
      base = ebase + ch * _K
      pltpu.async_copy(dst_hbm.at[pl.ds(base, _K)], dsts[p], isem[p])
      pltpu.async_copy(ec_hbm.at[pl.ds(base, _K)], facs[p], isem[p])

    def _wait_idx(ch, p):
      base = ebase + ch * _K
      pltpu.make_async_copy(
          dst_hbm.at[pl.ds(base, _K)], dsts[p], isem[p]).wait()
      pltpu.make_async_copy(
          ec_hbm.at[pl.ds(base, _K)], facs[p], isem[p]).wait()

    def _compute_and_write(ch, p):
      dst_p, fac_p = dsts[p], facs[p]
      def _facg(g, c2):
        d16 = dst_p[pl.ds(g * 16, 16)]
        c16 = plsc.load_gather(cntf_v, [d16])
        e16 = fac_p[pl.ds(g * 16, 16)]
        fac_p[pl.ds(g * 16, 16)] = e16 / jnp.maximum(c16, 1.0)
        return c2
      lax.fori_loop(0, _K // 16, _facg, 0)
      pltpu.sync_copy(facs[p], fac_out.at[pl.ds(ebase + ch * _K, _K)])

    _start_idx(0, 0)
    _start_idx(1, 1)

    def _fiter(i, c):
      for b in range(4):
        ch = 4 * i + b
        p2 = (b + 2) % 4
        _wait_idx(ch, b)
        _start_idx(ch + 2, p2)
        _compute_and_write(ch, b)
      return c
    lax.fori_loop(0, _NSUP, _fiter, 0)

    _wait_idx(_NCHUNK - 1, 0)
    _compute_and_write(_NCHUNK - 1, 0)
    _wait_idx(_NCHUNK, 1)

  return pl.kernel(
      body, out_type=out_type, mesh=_sc_mesh(), scratch_types=scratch,
      compiler_params=pltpu.CompilerParams(needs_layout_passes=False))


@functools.cache
def _make_sc_next():
  out_type = jax.ShapeDtypeStruct((_NCORE, _NPAD, _D), jnp.float32)

  scratch = [pltpu.VMEM_SHARED((_NPAD, _D), jnp.float32)]
  scratch += [pltpu.VMEM((_K, _D), jnp.float32) for _ in range(_RING)]
  scratch += [pltpu.VMEM((_K,), jnp.int32) for _ in range(_RING)]
  scratch += [pltpu.VMEM((_K,), jnp.int32) for _ in range(_RING)]
  scratch += [pltpu.VMEM((_K,), jnp.float32) for _ in range(_RING)]
  scratch += [pltpu.SemaphoreType.DMA for _ in range(3 * _RING)]

  def body(y_hbm, src_hbm, dst_hbm, fac_hbm, agg_out, *refs):
    rest = list(refs)
    accum_sh = rest.pop(0)
    rows = [rest.pop(0) for _ in range(_RING)]
    srcs = [rest.pop(0) for _ in range(_RING)]
    dsts = [rest.pop(0) for _ in range(_RING)]
    facs = [rest.pop(0) for _ in range(_RING)]
    isem = [rest.pop(0) for _ in range(_RING)]
    gsem = [rest.pop(0) for _ in range(_RING)]
    ssem = [rest.pop(0) for _ in range(_RING)]

    cid = lax.axis_index("c")
    sid = lax.axis_index("s")
    wid = sid * _NCORE + cid
    ebase = wid * _EPW

    z16 = jnp.zeros((16,), jnp.float32)
    zi16 = jnp.zeros((16,), jnp.int32)

    def _zr(i, c):
      for cc in range(_D // 16):
        rows[0][i, pl.ds(cc * 16, 16)] = z16
        rows[2][i, pl.ds(cc * 16, 16)] = z16
        rows[3][i, pl.ds(cc * 16, 16)] = z16
      return c
    lax.fori_loop(0, _K, _zr, 0)
    for g in range(_K // 16):
      dsts[2][pl.ds(g * 16, 16)] = zi16
      dsts[3][pl.ds(g * 16, 16)] = zi16
    for b in range(_RPT // _K):
      pltpu.sync_copy(rows[0], accum_sh.at[pl.ds(sid * _RPT + b * _K, _K)])

    plsc.subcore_barrier()

    def _start_idx(ch, p):
      base = ebase + ch * _K
      pltpu.async_copy(src_hbm.at[pl.ds(base, _K)], srcs[p], isem[p])
      pltpu.async_copy(dst_hbm.at[pl.ds(base, _K)], dsts[p], isem[p])
      pltpu.async_copy(fac_hbm.at[pl.ds(base, _K)], facs[p], isem[p])

    def _wait_idx(ch, p):
      base = ebase + ch * _K
      pltpu.make_async_copy(
          src_hbm.at[pl.ds(base, _K)], srcs[p], isem[p]).wait()
      pltpu.make_async_copy(
          dst_hbm.at[pl.ds(base, _K)], dsts[p], isem[p]).wait()
      pltpu.make_async_copy(
          fac_hbm.at[pl.ds(base, _K)], facs[p], isem[p]).wait()

    def _start_gather(p):
      pltpu.async_copy(y_hbm.at[srcs[p]], rows[p], gsem[p])

    def _start_scatter(p):
      pltpu.async_copy(rows[p], accum_sh.at[dsts[p]], ssem[p], add=True)

    def _wait_scatter(p):
      pltpu.make_async_copy(rows[p], accum_sh.at[dsts[p]], ssem[p]).wait()

    def _process(ch, p):
      pltpu.make_async_copy(y_hbm.at[srcs[p]], rows[p], gsem[p]).wait()
      rows_p, fac_p = rows[p], facs[p]
      def _scg(g, c2):
        for j in range(16):
          e = g * 16 + j
          bvec = plsc.load_gather(fac_p, [jnp.full((16,), e, jnp.int32)])
          for cc in range(_D // 16):
            rows_p[e, pl.ds(cc * 16, 16)] = (
                rows_p[e, pl.ds(cc * 16, 16)] * bvec)
        return c2
      lax.fori_loop(0, _K // 16, _scg, 0)
      _start_scatter(p)

    _start_idx(0, 0)
    _wait_idx(0, 0)
    _start_gather(0)
    _start_idx(1, 1)
    _start_scatter(2)
    _start_scatter(3)

    def _miter(i, c):
      for b in range(_RING):
        ch = _RING * i + b
        p1 = (b + 1) % _RING
        p2 = (b + 2) % _RING
        _wait_idx(ch + 1, p1)
        _start_gather(p1)
        _wait_scatter(p2)
        _start_idx(ch + 2, p2)
        _process(ch, b)
      return c
    lax.fori_loop(0, _NSUP, _miter, 0)

    _wait_scatter(2)
    _process(_NCHUNK - 1, 0)
    _wait_idx(_NCHUNK + 1, 1)
    _wait_scatter(3)
    _wait_scatter(0)

    plsc.subcore_barrier()
    pltpu.sync_copy(accum_sh.at[pl.ds(sid * _RPT, _RPT)],
                    agg_out.at[cid, pl.ds(sid * _RPT, _RPT)])

  return pl.kernel(
      body, out_type=out_type, mesh=_sc_mesh(), scratch_types=scratch,
      compiler_params=pltpu.CompilerParams(needs_layout_passes=False))


_BLK = 1024


def _layer_body(p_ref, x_ref, nc_ref, wl_ref, wr_ref, bl_ref, br_ref, o_ref):
  agg = p_ref[0] + p_ref[1]
  xr = x_ref[...] * nc_ref[...]
  acc = jnp.dot(agg, wl_ref[...], preferred_element_type=jnp.float32)
  acc = acc + jnp.dot(xr, wr_ref[...], preferred_element_type=jnp.float32)
  o_ref[...] = jnp.maximum(acc + bl_ref[...] + br_ref[...], 0.0)


def _layer_tc(p, xp, ncp, wlT, wrT, bl2, br2, interpret=False):
  return pl.pallas_call(
      _layer_body,
      grid=(_NPAD // _BLK,),
      in_specs=[
          pl.BlockSpec((_NCORE, _BLK, _D), lambda i: (0, i, 0)),
          pl.BlockSpec((_BLK, _D), lambda i: (i, 0)),
          pl.BlockSpec((_BLK, 1), lambda i: (i, 0)),
          pl.BlockSpec((_D, _D), lambda i: (0, 0)),
          pl.BlockSpec((_D, _D), lambda i: (0, 0)),
          pl.BlockSpec((1, _D), lambda i: (0, 0)),
          pl.BlockSpec((1, _D), lambda i: (0, 0)),
      ],
      out_specs=pl.BlockSpec((_BLK, _D), lambda i: (i, 0)),
      out_shape=jax.ShapeDtypeStruct((_NPAD, _D), jnp.float32),
      interpret=interpret,
  )(p, xp, ncp, wlT, wrT, bl2, br2)


def _pool_body(h_ref, b_ref, wc_ref, bc_ref, o_ref, ps_ref, pc_ref):
  i = pl.program_id(0)

  @pl.when(i == 0)
  def _():
    ps_ref[...] = jnp.zeros_like(ps_ref)
    pc_ref[...] = jnp.zeros_like(pc_ref)

  bvec = b_ref[...]
  iota = lax.broadcasted_iota(jnp.int32, (_G, _BLK), 0)
  oh = (iota == bvec).astype(jnp.float32)
  ps_ref[...] += jnp.dot(oh, h_ref[...], preferred_element_type=jnp.float32)
  pc_ref[...] += jnp.sum(oh, axis=1, keepdims=True)

  @pl.when(i == _NPAD // _BLK - 1)
  def _():
    pooled = ps_ref[...] / jnp.maximum(pc_ref[...], 1.0)
    o_ref[...] = (jnp.dot(pooled, wc_ref[...],
                          preferred_element_type=jnp.float32) + bc_ref[...])


def _pool_tc(h, batch2, wcT, bc2, interpret=False):
  return pl.pallas_call(
      _pool_body,
      grid=(_NPAD // _BLK,),
      in_specs=[
          pl.BlockSpec((_BLK, _D), lambda i: (i, 0)),
          pl.BlockSpec((1, _BLK), lambda i: (0, i)),
          pl.BlockSpec((_D, _OUT), lambda i: (0, 0)),
          pl.BlockSpec((1, _OUT), lambda i: (0, 0)),
      ],
      out_specs=pl.BlockSpec((_G, _OUT), lambda i: (0, 0)),
      out_shape=jax.ShapeDtypeStruct((_G, _OUT), jnp.float32),
      scratch_shapes=[
          pltpu.VMEM((_G, _D), jnp.float32),
          pltpu.VMEM((_G, 1), jnp.float32),
      ],
      interpret=interpret,
  )(h, batch2, wcT, bc2)


def kernel(x, edge_index, batch, node_centrality, edge_centrality,
           W_l0, b_l0, W_r0, b_r0, W_l1, b_l1, W_r1, b_r1, Wc, bc):
  f32 = jnp.float32
  xp = jnp.zeros((_NPAD, _D), f32).at[:_N].set(x)
  ncp = jnp.zeros((_NPAD, 1), f32).at[:_N, 0].set(node_centrality)
  batch2 = jnp.full((1, _NPAD), _G, jnp.int32).at[0, :_N].set(batch)

  src = edge_index[0]
  dst = edge_index[1]
  zpad_i = jnp.zeros((_EPAD,), jnp.int32)
  srcp = jnp.concatenate([src, zpad_i])
  dstp = jnp.concatenate([dst, zpad_i])

  ecp = jnp.concatenate([edge_centrality, jnp.zeros((_EPAD,), f32)])
  facp = _make_sc_fac()(dstp, ecp)
  p0 = _make_sc_next()(xp, srcp, dstp, facp)
  h1 = _layer_tc(p0, xp, ncp, W_l0.T, W_r0.T,
                 b_l0.reshape(1, -1), b_r0.reshape(1, -1))
  p1 = _make_sc_next()(h1, srcp, dstp, facp)
  h2 = _layer_tc(p1, h1, ncp, W_l1.T, W_r1.T,
                 b_l1.reshape(1, -1), b_r1.reshape(1, -1))
  return _pool_tc(h2, batch2, Wc.T, bc.reshape(1, -1))


# speedup vs baseline: 9.8868x; 1.0239x over previous
import functools

import jax
import jax.numpy as jnp
from jax import lax
from jax.experimental import pallas as pl
from jax.experimental.pallas import tpu as pltpu
from jax.experimental.pallas import tpu_sc as plsc

_N = 10000
_E = 320000
_D = 128
_G = 128
_OUT = 32
_NPAD = 10240

_NCORE = 2
_NSUB = 16
_NW = _NCORE * _NSUB
_EPW = _E // _NW
_K = 80
_NCHUNK = _EPW // _K
_RING = 4
_NSUP = (_NCHUNK - 1) // _RING
_EPAD = 8 * _K
_EPT = _E // _NSUB
_KC = 128
_NCC = _EPT // _KC
_KCR = _EPT - _NCC * _KC
_RPT = _NPAD // _NSUB


def _sc_mesh():
  return plsc.VectorSubcoreMesh(
      core_axis_name="c", subcore_axis_name="s",
      num_cores=_NCORE, num_subcores=_NSUB)


@functools.cache
def _make_sc_fac():
  out_type = jax.ShapeDtypeStruct((_E + _EPAD,), jnp.float32)

  scratch = [pltpu.VMEM_SHARED((_NPAD,), jnp.float32)]
  scratch += [pltpu.VMEM((_NPAD,), jnp.float32)]
  scratch += [pltpu.VMEM((_K,), jnp.int32) for _ in range(4)]
  scratch += [pltpu.VMEM((_K,), jnp.float32) for _ in range(4)]
  scratch += [pltpu.SemaphoreType.DMA for _ in range(4)]
  scratch += [
      pltpu.VMEM((_KC,), jnp.int32),
      pltpu.VMEM((_KC,), jnp.int32),
      pltpu.VMEM((_KCR,), jnp.int32),
      pltpu.VMEM((_KC,), jnp.float32),
      pltpu.VMEM((_RPT,), jnp.float32),
      pltpu.SemaphoreType.DMA,
      pltpu.SemaphoreType.DMA,
  ]

  def body(dst_hbm, ec_hbm, fac_out, *refs):
    rest = list(refs)
    cnt_sh = rest.pop(0)
    cntf_v = rest.pop(0)
    dsts = [rest.pop(0) for _ in range(4)]
    facs = [rest.pop(0) for _ in range(4)]
    isem = [rest.pop(0) for _ in range(4)]
    (dstc0, dstc1, dstr_v, ones_v, zflat_v, cs0, cs1) = rest

    cid = lax.axis_index("c")
    sid = lax.axis_index("s")
    wid = sid * _NCORE + cid

    z16 = jnp.zeros((16,), jnp.float32)
    one16 = jnp.ones((16,), jnp.float32)
    for g in range(_KC // 16):
      ones_v[pl.ds(g * 16, 16)] = one16
    def _zf(i, c):
      zflat_v[pl.ds(i * 16, 16)] = z16
      return c
    lax.fori_loop(0, _RPT // 16, _zf, 0)
    pltpu.sync_copy(zflat_v, cnt_sh.at[pl.ds(sid * _RPT, _RPT)])
    plsc.subcore_barrier()

    cbase = sid * _EPT
    def _cload(k, buf, sem):
      pltpu.async_copy(dst_hbm.at[pl.ds(cbase + k * _KC, _KC)], buf, sem)
    def _cwait(k, buf, sem):
      pltpu.make_async_copy(
          dst_hbm.at[pl.ds(cbase + k * _KC, _KC)], buf, sem).wait()
    _cload(0, dstc0, cs0)
    _cload(1, dstc1, cs1)
    def _citer(i, c):
      k0 = 2 * i
      _cwait(k0, dstc0, cs0)
      pltpu.sync_copy(ones_v, cnt_sh.at[dstc0], add=True)
      _cload(k0 + 2, dstc0, cs0)
      _cwait(k0 + 1, dstc1, cs1)
      pltpu.sync_copy(ones_v, cnt_sh.at[dstc1], add=True)
      _cload(k0 + 3, dstc1, cs1)
      return c
    lax.fori_loop(0, _NCC // 2, _citer, 0)
    _cwait(_NCC, dstc0, cs0)
    _cwait(_NCC + 1, dstc1, cs1)
    pltpu.sync_copy(dst_hbm.at[pl.ds(cbase + _NCC * _KC, _KCR)], dstr_v)
    pltpu.sync_copy(ones_v.at[pl.ds(0, _KCR)], cnt_sh.at[dstr_v], add=True)
    plsc.subcore_barrier()
    pltpu.sync_copy(cnt_sh, cntf_v)

    ebase = wid * _EPW

    def _start_idx(ch, p):
      base = ebase + ch * _K
      pltpu.async_copy(dst_hbm.at[pl.ds(base, _K)], dsts[p], isem[p])
      pltpu.async_copy(ec_hbm.at[pl.ds(base, _K)], facs[p], isem[p])

    def _wait_idx(ch, p):
      base = ebase + ch * _K
      pltpu.make_async_copy(
          dst_hbm.at[pl.ds(base, _K)], dsts[p], isem[p]).wait()
      pltpu.make_async_copy(
          ec_hbm.at[pl.ds(base, _K)], facs[p], isem[p]).wait()

    def _compute_and_write(ch, p):
      dst_p, fac_p = dsts[p], facs[p]
      def _facg(g, c2):
        d16 = dst_p[pl.ds(g * 16, 16)]
        c16 = plsc.load_gather(cntf_v, [d16])
        e16 = fac_p[pl.ds(g * 16, 16)]
        fac_p[pl.ds(g * 16, 16)] = e16 / jnp.maximum(c16, 1.0)
        return c2
      lax.fori_loop(0, _K // 16, _facg, 0)
      pltpu.sync_copy(facs[p], fac_out.at[pl.ds(ebase + ch * _K, _K)])

    _start_idx(0, 0)
    _start_idx(1, 1)

    def _fiter(i, c):
      for b in range(4):
        ch = 4 * i + b
        p2 = (b + 2) % 4
        _wait_idx(ch, b)
        _start_idx(ch + 2, p2)
        _compute_and_write(ch, b)
      return c
    lax.fori_loop(0, _NSUP, _fiter, 0)

    _wait_idx(_NCHUNK - 1, 0)
    _compute_and_write(_NCHUNK - 1, 0)
    _wait_idx(_NCHUNK, 1)

  return pl.kernel(
      body, out_type=out_type, mesh=_sc_mesh(), scratch_types=scratch,
      compiler_params=pltpu.CompilerParams(needs_layout_passes=False))


@functools.cache
def _make_sc_next():
  out_type = jax.ShapeDtypeStruct((_NCORE, _NPAD, _D), jnp.float32)

  scratch = [pltpu.VMEM_SHARED((_NPAD, _D), jnp.float32)]
  scratch += [pltpu.VMEM((_K, _D), jnp.float32) for _ in range(_RING)]
  scratch += [pltpu.VMEM((_K,), jnp.int32) for _ in range(_RING)]
  scratch += [pltpu.VMEM((_K,), jnp.int32) for _ in range(_RING)]
  scratch += [pltpu.VMEM((_K,), jnp.float32) for _ in range(_RING)]
  scratch += [pltpu.SemaphoreType.DMA for _ in range(3 * _RING)]

  def body(y_hbm, src_hbm, dst_hbm, fac_hbm, agg_out, *refs):
    rest = list(refs)
    accum_sh = rest.pop(0)
    rows = [rest.pop(0) for _ in range(_RING)]
    srcs = [rest.pop(0) for _ in range(_RING)]
    dsts = [rest.pop(0) for _ in range(_RING)]
    facs = [rest.pop(0) for _ in range(_RING)]
    isem = [rest.pop(0) for _ in range(_RING)]
    gsem = [rest.pop(0) for _ in range(_RING)]
    ssem = [rest.pop(0) for _ in range(_RING)]

    cid = lax.axis_index("c")
    sid = lax.axis_index("s")
    wid = sid * _NCORE + cid
    ebase = wid * _EPW

    z16 = jnp.zeros((16,), jnp.float32)
    zi16 = jnp.zeros((16,), jnp.int32)

    def _zr(i, c):
      for cc in range(_D // 16):
        rows[0][i, pl.ds(cc * 16, 16)] = z16
        rows[2][i, pl.ds(cc * 16, 16)] = z16
        rows[3][i, pl.ds(cc * 16, 16)] = z16
      return c
    lax.fori_loop(0, _K, _zr, 0)
    for g in range(_K // 16):
      dsts[2][pl.ds(g * 16, 16)] = zi16
      dsts[3][pl.ds(g * 16, 16)] = zi16
    for b in range(_RPT // _K):
      pltpu.sync_copy(rows[0], accum_sh.at[pl.ds(sid * _RPT + b * _K, _K)])

    plsc.subcore_barrier()

    def _start_idx(ch, p):
      base = ebase + ch * _K
      pltpu.async_copy(src_hbm.at[pl.ds(base, _K)], srcs[p], isem[p])
      pltpu.async_copy(dst_hbm.at[pl.ds(base, _K)], dsts[p], isem[p])
      pltpu.async_copy(fac_hbm.at[pl.ds(base, _K)], facs[p], isem[p])

    def _wait_idx(ch, p):
      base = ebase + ch * _K
      pltpu.make_async_copy(
          src_hbm.at[pl.ds(base, _K)], srcs[p], isem[p]).wait()
      pltpu.make_async_copy(
          dst_hbm.at[pl.ds(base, _K)], dsts[p], isem[p]).wait()
      pltpu.make_async_copy(
          fac_hbm.at[pl.ds(base, _K)], facs[p], isem[p]).wait()

    def _start_gather(p):
      pltpu.async_copy(y_hbm.at[srcs[p]], rows[p], gsem[p])

    def _start_scatter(p):
      pltpu.async_copy(rows[p], accum_sh.at[dsts[p]], ssem[p], add=True)

    def _wait_scatter(p):
      pltpu.make_async_copy(rows[p], accum_sh.at[dsts[p]], ssem[p]).wait()

    def _process(ch, p):
      pltpu.make_async_copy(y_hbm.at[srcs[p]], rows[p], gsem[p]).wait()
      rows_p, fac_p = rows[p], facs[p]
      def _scg(g, c2):
        for j in range(16):
          e = g * 16 + j
          bvec = plsc.load_gather(fac_p, [jnp.full((16,), e, jnp.int32)])
          for cc in range(_D // 16):
            rows_p[e, pl.ds(cc * 16, 16)] = (
                rows_p[e, pl.ds(cc * 16, 16)] * bvec)
        return c2
      lax.fori_loop(0, _K // 16, _scg, 0)
      _start_scatter(p)

    _start_idx(0, 0)
    _wait_idx(0, 0)
    _start_gather(0)
    _start_idx(1, 1)
    _start_scatter(2)
    _start_scatter(3)

    def _miter(i, c):
      for b in range(_RING):
        ch = _RING * i + b
        p1 = (b + 1) % _RING
        p2 = (b + 2) % _RING
        _wait_idx(ch + 1, p1)
        _start_gather(p1)
        _wait_scatter(p2)
        _start_idx(ch + 2, p2)
        _process(ch, b)
      return c
    lax.fori_loop(0, _NSUP, _miter, 0)

    _wait_scatter(2)
    _process(_NCHUNK - 1, 0)
    _wait_idx(_NCHUNK + 1, 1)
    _wait_scatter(3)
    _wait_scatter(0)

    plsc.subcore_barrier()
    pltpu.sync_copy(accum_sh.at[pl.ds(sid * _RPT, _RPT)],
                    agg_out.at[cid, pl.ds(sid * _RPT, _RPT)])

  return pl.kernel(
      body, out_type=out_type, mesh=_sc_mesh(), scratch_types=scratch,
      compiler_params=pltpu.CompilerParams(needs_layout_passes=False))


_BLK = 1024


def _xr_body(x_ref, nc_ref, wr_ref, br_ref, o_ref):
  xr = x_ref[...] * nc_ref[...]
  o_ref[...] = (jnp.dot(xr, wr_ref[...], preferred_element_type=jnp.float32)
                + br_ref[...])


def _xr_tc(xp, ncp, wrT, br2):
  return pl.pallas_call(
      _xr_body,
      grid=(_NPAD // _BLK,),
      in_specs=[
          pl.BlockSpec((_BLK, _D), lambda i: (i, 0)),
          pl.BlockSpec((_BLK, 1), lambda i: (i, 0)),
          pl.BlockSpec((_D, _D), lambda i: (0, 0)),
          pl.BlockSpec((1, _D), lambda i: (0, 0)),
      ],
      out_specs=pl.BlockSpec((_BLK, _D), lambda i: (i, 0)),
      out_shape=jax.ShapeDtypeStruct((_NPAD, _D), jnp.float32),
  )(xp, ncp, wrT, br2)


def _layer_body(p_ref, xr_ref, wl_ref, bl_ref, o_ref):
  agg = p_ref[0] + p_ref[1]
  acc = jnp.dot(agg, wl_ref[...], preferred_element_type=jnp.float32)
  o_ref[...] = jnp.maximum(acc + bl_ref[...] + xr_ref[...], 0.0)


def _layer_tc(p, xr, wlT, bl2, interpret=False):
  return pl.pallas_call(
      _layer_body,
      grid=(_NPAD // _BLK,),
      in_specs=[
          pl.BlockSpec((_NCORE, _BLK, _D), lambda i: (0, i, 0)),
          pl.BlockSpec((_BLK, _D), lambda i: (i, 0)),
          pl.BlockSpec((_D, _D), lambda i: (0, 0)),
          pl.BlockSpec((1, _D), lambda i: (0, 0)),
      ],
      out_specs=pl.BlockSpec((_BLK, _D), lambda i: (i, 0)),
      out_shape=jax.ShapeDtypeStruct((_NPAD, _D), jnp.float32),
      interpret=interpret,
  )(p, xr, wlT, bl2)


def _layer_pool_body(p_ref, xr_ref, wl_ref, bl_ref, b_ref, wc_ref, bc_ref,
                     o_ref, ps_ref, pc_ref):
  i = pl.program_id(0)

  @pl.when(i == 0)
  def _():
    ps_ref[...] = jnp.zeros_like(ps_ref)
    pc_ref[...] = jnp.zeros_like(pc_ref)

  agg = p_ref[0] + p_ref[1]
  acc = jnp.dot(agg, wl_ref[...], preferred_element_type=jnp.float32)
  h = jnp.maximum(acc + bl_ref[...] + xr_ref[...], 0.0)

  bvec = b_ref[...]
  iota = lax.broadcasted_iota(jnp.int32, (_G, _BLK), 0)
  oh = (iota == bvec).astype(jnp.float32)
  ps_ref[...] += jnp.dot(oh, h, preferred_element_type=jnp.float32)
  pc_ref[...] += jnp.sum(oh, axis=1, keepdims=True)

  @pl.when(i == _NPAD // _BLK - 1)
  def _():
    pooled = ps_ref[...] / jnp.maximum(pc_ref[...], 1.0)
    o_ref[...] = (jnp.dot(pooled, wc_ref[...],
                          preferred_element_type=jnp.float32) + bc_ref[...])


def _layer_pool_tc(p, xr, wlT, bl2, batch2, wcT, bc2):
  return pl.pallas_call(
      _layer_pool_body,
      grid=(_NPAD // _BLK,),
      in_specs=[
          pl.BlockSpec((_NCORE, _BLK, _D), lambda i: (0, i, 0)),
          pl.BlockSpec((_BLK, _D), lambda i: (i, 0)),
          pl.BlockSpec((_D, _D), lambda i: (0, 0)),
          pl.BlockSpec((1, _D), lambda i: (0, 0)),
          pl.BlockSpec((1, _BLK), lambda i: (0, i)),
          pl.BlockSpec((_D, _OUT), lambda i: (0, 0)),
          pl.BlockSpec((1, _OUT), lambda i: (0, 0)),
      ],
      out_specs=pl.BlockSpec((_G, _OUT), lambda i: (0, 0)),
      out_shape=jax.ShapeDtypeStruct((_G, _OUT), jnp.float32),
      scratch_shapes=[
          pltpu.VMEM((_G, _D), jnp.float32),
          pltpu.VMEM((_G, 1), jnp.float32),
      ],
  )(p, xr, wlT, bl2, batch2, wcT, bc2)


def kernel(x, edge_index, batch, node_centrality, edge_centrality,
           W_l0, b_l0, W_r0, b_r0, W_l1, b_l1, W_r1, b_r1, Wc, bc):
  f32 = jnp.float32
  xp = jnp.zeros((_NPAD, _D), f32).at[:_N].set(x)
  ncp = jnp.zeros((_NPAD, 1), f32).at[:_N, 0].set(node_centrality)
  batch2 = jnp.full((1, _NPAD), _G, jnp.int32).at[0, :_N].set(batch)

  src = edge_index[0]
  dst = edge_index[1]
  zpad_i = jnp.zeros((_EPAD,), jnp.int32)
  srcp = jnp.concatenate([src, zpad_i])
  dstp = jnp.concatenate([dst, zpad_i])

  ecp = jnp.concatenate([edge_centrality, jnp.zeros((_EPAD,), f32)])
  facp = _make_sc_fac()(dstp, ecp)
  xr1 = _xr_tc(xp, ncp, W_r0.T, b_r0.reshape(1, -1))
  p0 = _make_sc_next()(xp, srcp, dstp, facp)
  h1 = _layer_tc(p0, xr1, W_l0.T, b_l0.reshape(1, -1))
  xr2 = _xr_tc(h1, ncp, W_r1.T, b_r1.reshape(1, -1))
  p1 = _make_sc_next()(h1, srcp, dstp, facp)
  return _layer_pool_tc(p1, xr2, W_l1.T, b_l1.reshape(1, -1),
                        batch2, Wc.T, bc.reshape(1, -1))


# speedup vs baseline: 10.8727x; 1.0997x over previous
import functools

import jax
import jax.numpy as jnp
from jax import lax
from jax.experimental import pallas as pl
from jax.experimental.pallas import tpu as pltpu
from jax.experimental.pallas import tpu_sc as plsc

_N = 10000
_E = 320000
_D = 128
_G = 128
_OUT = 32
_NPAD = 10240

_NCORE = 2
_NSUB = 16
_NW = _NCORE * _NSUB
_EPW = _E // _NW
_K = 80
_NCHUNK = _EPW // _K
_RING = 4
_NSUP = (_NCHUNK - 1) // _RING
_EPAD = 8 * _K
_EPT = _E // _NSUB
_KC = 128
_NCC = _EPT // _KC
_KCR = _EPT - _NCC * _KC
_RPT = _NPAD // _NSUB


def _sc_mesh():
  return plsc.VectorSubcoreMesh(
      core_axis_name="c", subcore_axis_name="s",
      num_cores=_NCORE, num_subcores=_NSUB)


@functools.cache
def _make_sc_fac():
  out_type = jax.ShapeDtypeStruct((_E + _EPAD,), jnp.float32)

  scratch = [pltpu.VMEM_SHARED((_NPAD,), jnp.float32)]
  scratch += [pltpu.VMEM((_NPAD,), jnp.float32)]
  scratch += [pltpu.VMEM((_K,), jnp.int32) for _ in range(4)]
  scratch += [pltpu.VMEM((_K,), jnp.float32) for _ in range(4)]
  scratch += [pltpu.SemaphoreType.DMA for _ in range(4)]
  scratch += [
      pltpu.VMEM((_KC,), jnp.int32),
      pltpu.VMEM((_KC,), jnp.int32),
      pltpu.VMEM((_KCR,), jnp.int32),
      pltpu.VMEM((_KC,), jnp.float32),
      pltpu.VMEM((_RPT,), jnp.float32),
      pltpu.SemaphoreType.DMA,
      pltpu.SemaphoreType.DMA,
  ]

  def body(dst_hbm, ec_hbm, fac_out, *refs):
    rest = list(refs)
    cnt_sh = rest.pop(0)
    cntf_v = rest.pop(0)
    dsts = [rest.pop(0) for _ in range(4)]
    facs = [rest.pop(0) for _ in range(4)]
    isem = [rest.pop(0) for _ in range(4)]
    (dstc0, dstc1, dstr_v, ones_v, zflat_v, cs0, cs1) = rest

    cid = lax.axis_index("c")
    sid = lax.axis_index("s")
    wid = sid * _NCORE + cid

    z16 = jnp.zeros((16,), jnp.float32)
    one16 = jnp.ones((16,), jnp.float32)
    for g in range(_KC // 16):
      ones_v[pl.ds(g * 16, 16)] = one16
    def _zf(i, c):
      zflat_v[pl.ds(i * 16, 16)] = z16
      return c
    lax.fori_loop(0, _RPT // 16, _zf, 0)
    pltpu.sync_copy(zflat_v, cnt_sh.at[pl.ds(sid * _RPT, _RPT)])
    plsc.subcore_barrier()

    cbase = sid * _EPT
    def _cload(k, buf, sem):
      pltpu.async_copy(dst_hbm.at[pl.ds(cbase + k * _KC, _KC)], buf, sem)
    def _cwait(k, buf, sem):
      pltpu.make_async_copy(
          dst_hbm.at[pl.ds(cbase + k * _KC, _KC)], buf, sem).wait()
    _cload(0, dstc0, cs0)
    _cload(1, dstc1, cs1)
    def _citer(i, c):
      k0 = 2 * i
      _cwait(k0, dstc0, cs0)
      pltpu.sync_copy(ones_v, cnt_sh.at[dstc0], add=True)
      _cload(k0 + 2, dstc0, cs0)
      _cwait(k0 + 1, dstc1, cs1)
      pltpu.sync_copy(ones_v, cnt_sh.at[dstc1], add=True)
      _cload(k0 + 3, dstc1, cs1)
      return c
    lax.fori_loop(0, _NCC // 2, _citer, 0)
    _cwait(_NCC, dstc0, cs0)
    _cwait(_NCC + 1, dstc1, cs1)
    pltpu.sync_copy(dst_hbm.at[pl.ds(cbase + _NCC * _KC, _KCR)], dstr_v)
    pltpu.sync_copy(ones_v.at[pl.ds(0, _KCR)], cnt_sh.at[dstr_v], add=True)
    plsc.subcore_barrier()
    pltpu.sync_copy(cnt_sh, cntf_v)

    ebase = wid * _EPW

    def _start_idx(ch, p):
      base = ebase + ch * _K
      pltpu.async_copy(dst_hbm.at[pl.ds(base, _K)], dsts[p], isem[p])
      pltpu.async_copy(ec_hbm.at[pl.ds(base, _K)], facs[p], isem[p])

    def _wait_idx(ch, p):
      base = ebase + ch * _K
      pltpu.make_async_copy(
          dst_hbm.at[pl.ds(base, _K)], dsts[p], isem[p]).wait()
      pltpu.make_async_copy(
          ec_hbm.at[pl.ds(base, _K)], facs[p], isem[p]).wait()

    def _compute_and_write(ch, p):
      dst_p, fac_p = dsts[p], facs[p]
      def _facg(g, c2):
        d16 = dst_p[pl.ds(g * 16, 16)]
        c16 = plsc.load_gather(cntf_v, [d16])
        e16 = fac_p[pl.ds(g * 16, 16)]
        fac_p[pl.ds(g * 16, 16)] = e16 / jnp.maximum(c16, 1.0)
        return c2
      lax.fori_loop(0, _K // 16, _facg, 0)
      pltpu.sync_copy(facs[p], fac_out.at[pl.ds(ebase + ch * _K, _K)])

    _start_idx(0, 0)
    _start_idx(1, 1)

    def _fiter(i, c):
      for b in range(4):
        ch = 4 * i + b
        p2 = (b + 2) % 4
        _wait_idx(ch, b)
        _start_idx(ch + 2, p2)
        _compute_and_write(ch, b)
      return c
    lax.fori_loop(0, _NSUP, _fiter, 0)

    _wait_idx(_NCHUNK - 1, 0)
    _compute_and_write(_NCHUNK - 1, 0)
    _wait_idx(_NCHUNK, 1)

  return pl.kernel(
      body, out_type=out_type, mesh=_sc_mesh(), scratch_types=scratch,
      compiler_params=pltpu.CompilerParams(needs_layout_passes=False))


@functools.cache
def _make_sc_next():
  out_type = jax.ShapeDtypeStruct((_NCORE, _NPAD, _D), jnp.float32)

  scratch = [pltpu.VMEM_SHARED((_NPAD, _D), jnp.float32)]
  scratch += [pltpu.VMEM((_K, _D), jnp.float32) for _ in range(_RING)]
  scratch += [pltpu.VMEM((_K,), jnp.int32) for _ in range(_RING)]
  scratch += [pltpu.VMEM((_K,), jnp.int32) for _ in range(_RING)]
  scratch += [pltpu.VMEM((_K,), jnp.float32) for _ in range(_RING)]
  scratch += [pltpu.SemaphoreType.DMA for _ in range(3 * _RING)]

  def body(y_hbm, src_hbm, dst_hbm, fac_hbm, agg_out, *refs):
    rest = list(refs)
    accum_sh = rest.pop(0)
    rows = [rest.pop(0) for _ in range(_RING)]
    srcs = [rest.pop(0) for _ in range(_RING)]
    dsts = [rest.pop(0) for _ in range(_RING)]
    facs = [rest.pop(0) for _ in range(_RING)]
    isem = [rest.pop(0) for _ in range(_RING)]
    gsem = [rest.pop(0) for _ in range(_RING)]
    ssem = [rest.pop(0) for _ in range(_RING)]

    cid = lax.axis_index("c")
    sid = lax.axis_index("s")
    wid = sid * _NCORE + cid
    ebase = wid * _EPW

    z16 = jnp.zeros((16,), jnp.float32)
    zi16 = jnp.zeros((16,), jnp.int32)

    def _zr(i, c):
      for cc in range(_D // 16):
        rows[0][i, pl.ds(cc * 16, 16)] = z16
        rows[2][i, pl.ds(cc * 16, 16)] = z16
        rows[3][i, pl.ds(cc * 16, 16)] = z16
      return c
    lax.fori_loop(0, _K, _zr, 0)
    for g in range(_K // 16):
      dsts[2][pl.ds(g * 16, 16)] = zi16
      dsts[3][pl.ds(g * 16, 16)] = zi16
    for b in range(_RPT // _K):
      pltpu.sync_copy(rows[0], accum_sh.at[pl.ds(sid * _RPT + b * _K, _K)])

    plsc.subcore_barrier()

    def _start_idx(ch, p):
      base = ebase + ch * _K
      pltpu.async_copy(src_hbm.at[pl.ds(base, _K)], srcs[p], isem[p])
      pltpu.async_copy(dst_hbm.at[pl.ds(base, _K)], dsts[p], isem[p])
      pltpu.async_copy(fac_hbm.at[pl.ds(base, _K)], facs[p], isem[p])

    def _wait_idx(ch, p):
      base = ebase + ch * _K
      pltpu.make_async_copy(
          src_hbm.at[pl.ds(base, _K)], srcs[p], isem[p]).wait()
      pltpu.make_async_copy(
          dst_hbm.at[pl.ds(base, _K)], dsts[p], isem[p]).wait()
      pltpu.make_async_copy(
          fac_hbm.at[pl.ds(base, _K)], facs[p], isem[p]).wait()

    def _start_gather(p):
      pltpu.async_copy(y_hbm.at[srcs[p]], rows[p], gsem[p])

    def _start_scatter(p):
      pltpu.async_copy(rows[p], accum_sh.at[dsts[p]], ssem[p], add=True)

    def _wait_scatter(p):
      pltpu.make_async_copy(rows[p], accum_sh.at[dsts[p]], ssem[p]).wait()

    def _process(ch, p):
      pltpu.make_async_copy(y_hbm.at[srcs[p]], rows[p], gsem[p]).wait()
      rows_p, fac_p = rows[p], facs[p]
      def _scg(g, c2):
        f16 = fac_p[pl.ds(g * 16, 16)]
        for j in range(16):
          e = g * 16 + j
          bvec = lax.gather(
              f16, jnp.full((16, 1), j, jnp.int32),
              lax.GatherDimensionNumbers(offset_dims=(),
                                         collapsed_slice_dims=(0,),
                                         start_index_map=(0,)),
              (1,), mode=lax.GatherScatterMode.PROMISE_IN_BOUNDS)
          for cc in range(_D // 16):
            rows_p[e, pl.ds(cc * 16, 16)] = (
                rows_p[e, pl.ds(cc * 16, 16)] * bvec)
        return c2
      lax.fori_loop(0, _K // 16, _scg, 0)
      _start_scatter(p)

    _start_idx(0, 0)
    _wait_idx(0, 0)
    _start_gather(0)
    _start_idx(1, 1)
    _start_scatter(2)
    _start_scatter(3)

    def _miter(i, c):
      for b in range(_RING):
        ch = _RING * i + b
        p1 = (b + 1) % _RING
        p2 = (b + 2) % _RING
        _wait_idx(ch + 1, p1)
        _start_gather(p1)
        _wait_scatter(p2)
        _start_idx(ch + 2, p2)
        _process(ch, b)
      return c
    lax.fori_loop(0, _NSUP, _miter, 0)

    _wait_scatter(2)
    _process(_NCHUNK - 1, 0)
    _wait_idx(_NCHUNK + 1, 1)
    _wait_scatter(3)
    _wait_scatter(0)

    plsc.subcore_barrier()
    pltpu.sync_copy(accum_sh.at[pl.ds(sid * _RPT, _RPT)],
                    agg_out.at[cid, pl.ds(sid * _RPT, _RPT)])

  return pl.kernel(
      body, out_type=out_type, mesh=_sc_mesh(), scratch_types=scratch,
      compiler_params=pltpu.CompilerParams(needs_layout_passes=False))


_BLK = 1024


def _xr_body(x_ref, nc_ref, wr_ref, br_ref, o_ref):
  xr = x_ref[...] * nc_ref[...]
  o_ref[...] = (jnp.dot(xr, wr_ref[...], preferred_element_type=jnp.float32)
                + br_ref[...])


def _xr_tc(xp, ncp, wrT, br2):
  return pl.pallas_call(
      _xr_body,
      grid=(_NPAD // _BLK,),
      in_specs=[
          pl.BlockSpec((_BLK, _D), lambda i: (i, 0)),
          pl.BlockSpec((_BLK, 1), lambda i: (i, 0)),
          pl.BlockSpec((_D, _D), lambda i: (0, 0)),
          pl.BlockSpec((1, _D), lambda i: (0, 0)),
      ],
      out_specs=pl.BlockSpec((_BLK, _D), lambda i: (i, 0)),
      out_shape=jax.ShapeDtypeStruct((_NPAD, _D), jnp.float32),
  )(xp, ncp, wrT, br2)


def _layer_body(p_ref, xr_ref, wl_ref, bl_ref, o_ref):
  agg = p_ref[0] + p_ref[1]
  acc = jnp.dot(agg, wl_ref[...], preferred_element_type=jnp.float32)
  o_ref[...] = jnp.maximum(acc + bl_ref[...] + xr_ref[...], 0.0)


def _layer_tc(p, xr, wlT, bl2, interpret=False):
  return pl.pallas_call(
      _layer_body,
      grid=(_NPAD // _BLK,),
      in_specs=[
          pl.BlockSpec((_NCORE, _BLK, _D), lambda i: (0, i, 0)),
          pl.BlockSpec((_BLK, _D), lambda i: (i, 0)),
          pl.BlockSpec((_D, _D), lambda i: (0, 0)),
          pl.BlockSpec((1, _D), lambda i: (0, 0)),
      ],
      out_specs=pl.BlockSpec((_BLK, _D), lambda i: (i, 0)),
      out_shape=jax.ShapeDtypeStruct((_NPAD, _D), jnp.float32),
      interpret=interpret,
  )(p, xr, wlT, bl2)


def _layer_pool_body(p_ref, xr_ref, wl_ref, bl_ref, b_ref, wc_ref, bc_ref,
                     o_ref, ps_ref, pc_ref):
  i = pl.program_id(0)

  @pl.when(i == 0)
  def _():
    ps_ref[...] = jnp.zeros_like(ps_ref)
    pc_ref[...] = jnp.zeros_like(pc_ref)

  agg = p_ref[0] + p_ref[1]
  acc = jnp.dot(agg, wl_ref[...], preferred_element_type=jnp.float32)
  h = jnp.maximum(acc + bl_ref[...] + xr_ref[...], 0.0)

  bvec = b_ref[...]
  iota = lax.broadcasted_iota(jnp.int32, (_G, _BLK), 0)
  oh = (iota == bvec).astype(jnp.float32)
  ps_ref[...] += jnp.dot(oh, h, preferred_element_type=jnp.float32)
  pc_ref[...] += jnp.sum(oh, axis=1, keepdims=True)

  @pl.when(i == _NPAD // _BLK - 1)
  def _():
    pooled = ps_ref[...] / jnp.maximum(pc_ref[...], 1.0)
    o_ref[...] = (jnp.dot(pooled, wc_ref[...],
                          preferred_element_type=jnp.float32) + bc_ref[...])


def _layer_pool_tc(p, xr, wlT, bl2, batch2, wcT, bc2):
  return pl.pallas_call(
      _layer_pool_body,
      grid=(_NPAD // _BLK,),
      in_specs=[
          pl.BlockSpec((_NCORE, _BLK, _D), lambda i: (0, i, 0)),
          pl.BlockSpec((_BLK, _D), lambda i: (i, 0)),
          pl.BlockSpec((_D, _D), lambda i: (0, 0)),
          pl.BlockSpec((1, _D), lambda i: (0, 0)),
          pl.BlockSpec((1, _BLK), lambda i: (0, i)),
          pl.BlockSpec((_D, _OUT), lambda i: (0, 0)),
          pl.BlockSpec((1, _OUT), lambda i: (0, 0)),
      ],
      out_specs=pl.BlockSpec((_G, _OUT), lambda i: (0, 0)),
      out_shape=jax.ShapeDtypeStruct((_G, _OUT), jnp.float32),
      scratch_shapes=[
          pltpu.VMEM((_G, _D), jnp.float32),
          pltpu.VMEM((_G, 1), jnp.float32),
      ],
  )(p, xr, wlT, bl2, batch2, wcT, bc2)


def kernel(x, edge_index, batch, node_centrality, edge_centrality,
           W_l0, b_l0, W_r0, b_r0, W_l1, b_l1, W_r1, b_r1, Wc, bc):
  f32 = jnp.float32
  xp = jnp.zeros((_NPAD, _D), f32).at[:_N].set(x)
  ncp = jnp.zeros((_NPAD, 1), f32).at[:_N, 0].set(node_centrality)
  batch2 = jnp.full((1, _NPAD), _G, jnp.int32).at[0, :_N].set(batch)

  src = edge_index[0]
  dst = edge_index[1]
  zpad_i = jnp.zeros((_EPAD,), jnp.int32)
  srcp = jnp.concatenate([src, zpad_i])
  dstp = jnp.concatenate([dst, zpad_i])

  ecp = jnp.concatenate([edge_centrality, jnp.zeros((_EPAD,), f32)])
  facp = _make_sc_fac()(dstp, ecp)
  xr1 = _xr_tc(xp, ncp, W_r0.T, b_r0.reshape(1, -1))
  p0 = _make_sc_next()(xp, srcp, dstp, facp)
  h1 = _layer_tc(p0, xr1, W_l0.T, b_l0.reshape(1, -1))
  xr2 = _xr_tc(h1, ncp, W_r1.T, b_r1.reshape(1, -1))
  p1 = _make_sc_next()(h1, srcp, dstp, facp)
  return _layer_pool_tc(p1, xr2, W_l1.T, b_l1.reshape(1, -1),
                        batch2, Wc.T, bc.reshape(1, -1))
